# baseline (speedup 1.0000x reference)
import functools

import jax
import jax.numpy as jnp
import numpy as np
from jax.experimental import pallas as pl
from jax.experimental.pallas import tpu as pltpu

_K_NEIGHBORS = 16
_NEG = -1e30


def _hyper_kernel(x_ref, wq_ref, bq_ref, sel_ref, kp_ref, out_ref, *, t_dim):
    inter, m_dim = kp_ref.shape
    v_dim = sel_ref.shape[1]
    x2d = x_ref[0]
    hi = jax.lax.Precision.HIGHEST
    p = jnp.dot(wq_ref[...].astype(jnp.bfloat16), x2d.astype(jnp.bfloat16),
                preferred_element_type=jnp.float32)
    q = jnp.dot(p, sel_ref[...], preferred_element_type=jnp.float32,
                precision=hi)
    q = q * (1.0 / t_dim) + bq_ref[...]
    norm = jnp.sqrt(jnp.sum(q * q, axis=0, keepdims=True))
    qn = q / jnp.maximum(norm, 1e-12)
    scale = inter ** (-0.5)
    h = jax.lax.dot_general(
        qn.astype(jnp.bfloat16), kp_ref[...].astype(jnp.bfloat16),
        (((0,), (0,)), ((), ())),
        preferred_element_type=jnp.float32) * scale

    iota = jax.lax.broadcasted_iota(jnp.int32, (v_dim, m_dim), 1)
    work = h
    mask = jnp.zeros((v_dim, m_dim), jnp.bool_)
    rmax = jnp.max(h, axis=1, keepdims=True)
    for _ in range(_K_NEIGHBORS):
        mval = jnp.max(work, axis=1, keepdims=True)
        ismax = work == mval
        idx = jnp.min(jnp.where(ismax, iota, m_dim), axis=1, keepdims=True)
        sel1 = iota == idx
        mask = jnp.logical_or(mask, sel1)
        work = jnp.where(sel1, _NEG, work)
    ex = jnp.where(mask, jnp.exp(h - rmax), 0.0)
    denom = jnp.sum(ex, axis=1, keepdims=True)
    out_ref[0] = ex / denom


def kernel(x, W_q, b_q, key_prototypes):
    N, C, T, V = x.shape
    inter, M = key_prototypes.shape
    x2d = x.reshape(N, C, T * V)
    sel = (np.arange(T * V)[:, None] % V == np.arange(V)[None, :]).astype(np.float32)
    sel = jnp.asarray(sel)
    bq2 = b_q.reshape(inter, 1)

    grid = (N,)
    return pl.pallas_call(
        functools.partial(_hyper_kernel, t_dim=T),
        grid=grid,
        in_specs=[
            pl.BlockSpec((1, C, T * V), lambda i: (i, 0, 0)),
            pl.BlockSpec((inter, C), lambda i: (0, 0)),
            pl.BlockSpec((inter, 1), lambda i: (0, 0)),
            pl.BlockSpec((T * V, V), lambda i: (0, 0)),
            pl.BlockSpec((inter, M), lambda i: (0, 0)),
        ],
        out_specs=pl.BlockSpec((1, V, M), lambda i: (i, 0, 0)),
        out_shape=jax.ShapeDtypeStruct((N, V, M), jnp.float32),
        compiler_params=pltpu.CompilerParams(
            dimension_semantics=("arbitrary",),
        ),
    )(x2d, W_q, bq2, sel, key_prototypes)


# speedup vs baseline: 1.7987x; 1.7987x over previous
import functools

import jax
import jax.numpy as jnp
import numpy as np
from jax.experimental import pallas as pl
from jax.experimental.pallas import tpu as pltpu

_K_NEIGHBORS = 16
_NEG = -1e30
_BN = 8


def _hyper_kernel(x_ref, wq_ref, bq_ref, sel_ref, kp_ref, out_ref, *, t_dim):
    inter, m_dim = kp_ref.shape
    v_dim = sel_ref.shape[1]
    bn = x_ref.shape[0]
    hi = jax.lax.Precision.HIGHEST

    hs = []
    for b in range(bn):
        x2d = x_ref[b]
        p = jnp.dot(wq_ref[...], x2d, preferred_element_type=jnp.float32)
        q = jnp.dot(p, sel_ref[...], preferred_element_type=jnp.float32,
                    precision=hi)
        q = q * (1.0 / t_dim) + bq_ref[...]
        norm = jnp.sqrt(jnp.sum(q * q, axis=0, keepdims=True))
        qn = q / jnp.maximum(norm, 1e-12)
        h_b = jax.lax.dot_general(
            qn, kp_ref[...], (((0,), (0,)), ((), ())),
            preferred_element_type=jnp.float32) * (inter ** -0.5)
        hs.append(h_b[None])
    h = jnp.concatenate(hs, axis=0)

    iota = jax.lax.broadcasted_iota(jnp.int32, (bn, v_dim, m_dim), 2)
    work = h
    mask = jnp.zeros((bn, v_dim, m_dim), jnp.bool_)
    rmax = None
    for _ in range(_K_NEIGHBORS):
        mval = jnp.max(work, axis=2, keepdims=True)
        if rmax is None:
            rmax = mval
        ismax = work == mval
        idx = jnp.min(jnp.where(ismax, iota, m_dim), axis=2, keepdims=True)
        sel1 = iota == idx
        mask = jnp.logical_or(mask, sel1)
        work = jnp.where(sel1, _NEG, work)
    ex = jnp.where(mask, jnp.exp(h - rmax), 0.0)
    denom = jnp.sum(ex, axis=2, keepdims=True)
    out_ref[...] = ex / denom


def kernel(x, W_q, b_q, key_prototypes):
    N, C, T, V = x.shape
    inter, M = key_prototypes.shape
    x2d = x.reshape(N, C, T * V)
    sel = (np.arange(T * V)[:, None] % V == np.arange(V)[None, :]).astype(np.float32)
    sel = jnp.asarray(sel)
    bq2 = b_q.reshape(inter, 1)

    grid = (N // _BN,)
    return pl.pallas_call(
        functools.partial(_hyper_kernel, t_dim=T),
        grid=grid,
        in_specs=[
            pl.BlockSpec((_BN, C, T * V), lambda i: (i, 0, 0)),
            pl.BlockSpec((inter, C), lambda i: (0, 0)),
            pl.BlockSpec((inter, 1), lambda i: (0, 0)),
            pl.BlockSpec((T * V, V), lambda i: (0, 0)),
            pl.BlockSpec((inter, M), lambda i: (0, 0)),
        ],
        out_specs=pl.BlockSpec((_BN, V, M), lambda i: (i, 0, 0)),
        out_shape=jax.ShapeDtypeStruct((N, V, M), jnp.float32),
        compiler_params=pltpu.CompilerParams(
            dimension_semantics=("arbitrary",),
        ),
    )(x2d, W_q, bq2, sel, key_prototypes)
